# baseline (speedup 1.0000x reference)
import jax
import jax.numpy as jnp
from jax.experimental import pallas as pl
from jax.experimental.pallas import tpu as pltpu

EPS = 1e-5

_W = 64
_S = 64 * 64
_FLAT = _S + 2 * 128


def _conv_stats_kernel(x_ref, w3a_ref, w3b_ref, y_ref, stats_ref, p_ref):
    xb = x_ref[0].astype(jnp.bfloat16)
    cin = xb.shape[0]
    zpad = jnp.zeros((cin, 128), jnp.bfloat16)
    xf = jnp.concatenate([zpad, xb, zpad], axis=1)
    col = jax.lax.rem(jax.lax.broadcasted_iota(jnp.int32, (1, _FLAT), 1), _W)
    zero = jnp.zeros((), jnp.bfloat16)
    x_l = jnp.where(col == _W - 1, zero, xf)
    x_r = jnp.where(col == 0, zero, xf)

    def _roll(src, c):
        return jnp.concatenate([src[:, c:], src[:, :c]], axis=1)

    p_ref[0 * cin:1 * cin, :] = _roll(x_l, 127)
    p_ref[1 * cin:2 * cin, :] = _roll(x_l, 63)
    p_ref[2 * cin:3 * cin, :] = _roll(xf, 64)
    p_ref[3 * cin:4 * cin, :] = _roll(x_r, 65)
    y = jnp.dot(w3a_ref[...], p_ref[0:4 * cin, 0:_S],
                preferred_element_type=jnp.float32)
    p_ref[4 * cin:5 * cin, :] = xf
    p_ref[5 * cin:6 * cin, :] = _roll(x_r, 1)
    y = y + jnp.dot(w3b_ref[...], p_ref[1 * cin:6 * cin, 128:128 + _S],
                    preferred_element_type=jnp.float32)
    s1 = jnp.sum(y, axis=1, keepdims=True)
    s2 = jnp.sum(y * y, axis=1, keepdims=True)
    stats_ref[0] = jnp.concatenate([s1, s2], axis=1)
    y_ref[0] = y.astype(jnp.bfloat16)


def _norm_proj_kernel(y_ref, scale_ref, shift_ref, wseg_ref, bseg_ref,
                      out_ref):
    y = y_ref[0].astype(jnp.float32)
    z = jnp.maximum(y * scale_ref[...] + shift_ref[...], 0.0)
    out = jnp.dot(wseg_ref[...], z.astype(jnp.bfloat16),
                  preferred_element_type=jnp.float32)
    out_ref[0] = out + bseg_ref[...]


def kernel(x, w3_hwio, b3, gamma, beta, wseg_io, bseg):
    N, Cin, H, W = x.shape
    Cout = wseg_io.shape[1]
    S = H * W

    del b3

    xs = x.reshape(N, Cin, S)

    w3_t = jnp.transpose(w3_hwio, (3, 0, 1, 2)).reshape(Cin, 9, Cin)
    w3_t = w3_t.astype(jnp.bfloat16)
    w3_a = jnp.concatenate([w3_t[:, t] for t in (3, 0, 1, 2)], axis=1)
    w3_b = jnp.concatenate([w3_t[:, t] for t in (6, 7, 8, 4, 5)], axis=1)

    parallel = pltpu.CompilerParams(dimension_semantics=("parallel",))

    y_bf, stats = pl.pallas_call(
        _conv_stats_kernel,
        out_shape=(jax.ShapeDtypeStruct((N, Cin, S), jnp.bfloat16),
                   jax.ShapeDtypeStruct((N, Cin, 2), jnp.float32)),
        grid=(N,),
        in_specs=[
            pl.BlockSpec((1, Cin, S), lambda n: (n, 0, 0)),
            pl.BlockSpec((Cin, 4 * Cin), lambda n: (0, 0)),
            pl.BlockSpec((Cin, 5 * Cin), lambda n: (0, 0)),
        ],
        out_specs=(pl.BlockSpec((1, Cin, S), lambda n: (n, 0, 0)),
                   pl.BlockSpec((1, Cin, 2), lambda n: (n, 0, 0))),
        scratch_shapes=[pltpu.VMEM((6 * Cin, _FLAT), jnp.bfloat16)],
        compiler_params=parallel,
    )(xs, w3_a, w3_b)

    cnt = float(N * S)
    mean = jnp.sum(stats[:, :, 0], axis=0) / cnt
    var = jnp.maximum(jnp.sum(stats[:, :, 1], axis=0) / cnt - mean * mean,
                      0.0)
    scale = gamma * jax.lax.rsqrt(var + EPS)
    shift = beta - mean * scale

    out_hw = pl.pallas_call(
        _norm_proj_kernel,
        out_shape=jax.ShapeDtypeStruct((N, Cout, S), jnp.float32),
        grid=(N,),
        in_specs=[
            pl.BlockSpec((1, Cin, S), lambda n: (n, 0, 0)),
            pl.BlockSpec((Cin, 1), lambda n: (0, 0)),
            pl.BlockSpec((Cin, 1), lambda n: (0, 0)),
            pl.BlockSpec((Cout, Cin), lambda n: (0, 0)),
            pl.BlockSpec((Cout, 1), lambda n: (0, 0)),
        ],
        out_specs=pl.BlockSpec((1, Cout, S), lambda n: (n, 0, 0)),
        compiler_params=parallel,
    )(y_bf, scale.reshape(Cin, 1), shift.reshape(Cin, 1),
      jnp.transpose(wseg_io).astype(jnp.bfloat16), bseg.reshape(Cout, 1))

    return out_hw.reshape(N, Cout, H, W)


# speedup vs baseline: 1.1124x; 1.1124x over previous
import jax
import jax.numpy as jnp
from jax.experimental import pallas as pl
from jax.experimental.pallas import tpu as pltpu

EPS = 1e-5

_W = 64
_S = 64 * 64
_FLAT = _S + 2 * 128
_B = 2


def _conv_stats_kernel(x_ref, w3a_ref, w3b_ref, y_ref, stats_ref, p_ref):
    cin = x_ref.shape[1]
    xb = x_ref[...].reshape(_B * cin, _S).astype(jnp.bfloat16)
    zpad = jnp.zeros((_B * cin, 128), jnp.bfloat16)
    xf = jnp.concatenate([zpad, xb, zpad], axis=1)
    col = jax.lax.rem(jax.lax.broadcasted_iota(jnp.int32, (1, _FLAT), 1), _W)
    zero = jnp.zeros((), jnp.bfloat16)
    x_l = jnp.where(col == _W - 1, zero, xf)
    x_r = jnp.where(col == 0, zero, xf)

    def _roll(src, c):
        return jnp.concatenate([src[:, c:], src[:, :c]], axis=1)

    def _put(block, val):
        for i in range(_B):
            p_ref[i, block * cin:(block + 1) * cin, :] = \
                val[i * cin:(i + 1) * cin, :]

    _put(0, _roll(x_l, 127))
    _put(1, _roll(x_l, 63))
    _put(2, _roll(xf, 64))
    _put(3, _roll(x_r, 65))
    _put(4, xf)
    _put(5, _roll(x_r, 1))
    for i in range(_B):
        y = jnp.dot(w3a_ref[...], p_ref[i, 0:4 * cin, 0:_S],
                    preferred_element_type=jnp.float32)
        y = y + jnp.dot(w3b_ref[...], p_ref[i, 1 * cin:6 * cin, 128:128 + _S],
                        preferred_element_type=jnp.float32)
        s1 = jnp.sum(y, axis=1, keepdims=True)
        s2 = jnp.sum(y * y, axis=1, keepdims=True)
        stats_ref[i] = jnp.concatenate([s1, s2], axis=1)
        y_ref[i] = y.astype(jnp.bfloat16)


def _norm_proj_kernel(y_ref, scale_ref, shift_ref, wseg_ref, bseg_ref,
                      out_ref):
    cin = y_ref.shape[1]
    yb = y_ref[...].reshape(_B * cin, _S)
    sc = jnp.concatenate([scale_ref[...]] * _B, axis=0)
    sh = jnp.concatenate([shift_ref[...]] * _B, axis=0)
    z = jnp.maximum(yb.astype(jnp.float32) * sc + sh, 0.0)
    zb = z.astype(jnp.bfloat16)
    for i in range(_B):
        out = jnp.dot(wseg_ref[...], zb[i * cin:(i + 1) * cin, :],
                      preferred_element_type=jnp.float32)
        out_ref[i] = out + bseg_ref[...]


def kernel(x, w3_hwio, b3, gamma, beta, wseg_io, bseg):
    N, Cin, H, W = x.shape
    Cout = wseg_io.shape[1]
    S = H * W
    G = N // _B

    del b3

    xs = x.reshape(N, Cin, S)

    w3_t = jnp.transpose(w3_hwio, (3, 0, 1, 2)).reshape(Cin, 9, Cin)
    w3_t = w3_t.astype(jnp.bfloat16)
    w3_a = jnp.concatenate([w3_t[:, t] for t in (3, 0, 1, 2)], axis=1)
    w3_b = jnp.concatenate([w3_t[:, t] for t in (6, 7, 8, 4, 5)], axis=1)

    parallel = pltpu.CompilerParams(dimension_semantics=("parallel",))

    y_bf, stats = pl.pallas_call(
        _conv_stats_kernel,
        out_shape=(jax.ShapeDtypeStruct((N, Cin, S), jnp.bfloat16),
                   jax.ShapeDtypeStruct((N, Cin, 2), jnp.float32)),
        grid=(G,),
        in_specs=[
            pl.BlockSpec((_B, Cin, S), lambda n: (n, 0, 0)),
            pl.BlockSpec((Cin, 4 * Cin), lambda n: (0, 0)),
            pl.BlockSpec((Cin, 5 * Cin), lambda n: (0, 0)),
        ],
        out_specs=(pl.BlockSpec((_B, Cin, S), lambda n: (n, 0, 0)),
                   pl.BlockSpec((_B, Cin, 2), lambda n: (n, 0, 0))),
        scratch_shapes=[pltpu.VMEM((_B, 6 * Cin, _FLAT), jnp.bfloat16)],
        compiler_params=parallel,
    )(xs, w3_a, w3_b)

    cnt = float(N * S)
    mean = jnp.sum(stats[:, :, 0], axis=0) / cnt
    var = jnp.maximum(jnp.sum(stats[:, :, 1], axis=0) / cnt - mean * mean,
                      0.0)
    scale = gamma * jax.lax.rsqrt(var + EPS)
    shift = beta - mean * scale

    out_hw = pl.pallas_call(
        _norm_proj_kernel,
        out_shape=jax.ShapeDtypeStruct((N, Cout, S), jnp.float32),
        grid=(G,),
        in_specs=[
            pl.BlockSpec((_B, Cin, S), lambda n: (n, 0, 0)),
            pl.BlockSpec((Cin, 1), lambda n: (0, 0)),
            pl.BlockSpec((Cin, 1), lambda n: (0, 0)),
            pl.BlockSpec((Cout, Cin), lambda n: (0, 0)),
            pl.BlockSpec((Cout, 1), lambda n: (0, 0)),
        ],
        out_specs=pl.BlockSpec((_B, Cout, S), lambda n: (n, 0, 0)),
        compiler_params=parallel,
    )(y_bf, scale.reshape(Cin, 1), shift.reshape(Cin, 1),
      jnp.transpose(wseg_io).astype(jnp.bfloat16), bseg.reshape(Cout, 1))

    return out_hw.reshape(N, Cout, H, W)


# speedup vs baseline: 1.1565x; 1.0397x over previous
import jax
import jax.numpy as jnp
from jax.experimental import pallas as pl
from jax.experimental.pallas import tpu as pltpu

EPS = 1e-5

_W = 64
_S = 64 * 64
_FLAT = _S + 2 * 128
_B = 4


def _conv_stats_kernel(x_ref, w3a_ref, w3b_ref, y_ref, stats_ref, p_ref):
    cin = x_ref.shape[1]
    xb = x_ref[...].reshape(_B * cin, _S).astype(jnp.bfloat16)
    zpad = jnp.zeros((_B * cin, 128), jnp.bfloat16)
    xf = jnp.concatenate([zpad, xb, zpad], axis=1)
    col = jax.lax.rem(jax.lax.broadcasted_iota(jnp.int32, (1, _FLAT), 1), _W)
    zero = jnp.zeros((), jnp.bfloat16)
    x_l = jnp.where(col == _W - 1, zero, xf)
    x_r = jnp.where(col == 0, zero, xf)

    def _roll(src, c):
        return jnp.concatenate([src[:, c:], src[:, :c]], axis=1)

    def _put(block, val):
        for i in range(_B):
            p_ref[i, block * cin:(block + 1) * cin, :] = \
                val[i * cin:(i + 1) * cin, :]

    _put(0, _roll(x_l, 127))
    _put(1, _roll(x_l, 63))
    _put(2, _roll(xf, 64))
    _put(3, _roll(x_r, 65))
    _put(4, xf)
    _put(5, _roll(x_r, 1))
    for i in range(_B):
        y = jnp.dot(w3a_ref[...], p_ref[i, 0:4 * cin, 0:_S],
                    preferred_element_type=jnp.float32)
        y = y + jnp.dot(w3b_ref[...], p_ref[i, 1 * cin:6 * cin, 128:128 + _S],
                        preferred_element_type=jnp.float32)
        s1 = jnp.sum(y, axis=1, keepdims=True)
        s2 = jnp.sum(y * y, axis=1, keepdims=True)
        stats_ref[i] = jnp.concatenate([s1, s2], axis=1)
        y_ref[i] = y.astype(jnp.bfloat16)


def _norm_proj_kernel(y_ref, scale_ref, shift_ref, wseg_ref, bseg_ref,
                      out_ref):
    cin = y_ref.shape[1]
    yb = y_ref[...].reshape(_B * cin, _S)
    sc = jnp.concatenate([scale_ref[...]] * _B, axis=0)
    sh = jnp.concatenate([shift_ref[...]] * _B, axis=0)
    z = jnp.maximum(yb.astype(jnp.float32) * sc + sh, 0.0)
    zb = z.astype(jnp.bfloat16)
    for i in range(_B):
        out = jnp.dot(wseg_ref[...], zb[i * cin:(i + 1) * cin, :],
                      preferred_element_type=jnp.float32)
        out_ref[i] = out + bseg_ref[...]


def kernel(x, w3_hwio, b3, gamma, beta, wseg_io, bseg):
    N, Cin, H, W = x.shape
    Cout = wseg_io.shape[1]
    S = H * W
    G = N // _B

    del b3

    xs = x.reshape(N, Cin, S)

    w3_t = jnp.transpose(w3_hwio, (3, 0, 1, 2)).reshape(Cin, 9, Cin)
    w3_t = w3_t.astype(jnp.bfloat16)
    w3_a = jnp.concatenate([w3_t[:, t] for t in (3, 0, 1, 2)], axis=1)
    w3_b = jnp.concatenate([w3_t[:, t] for t in (6, 7, 8, 4, 5)], axis=1)

    parallel = pltpu.CompilerParams(dimension_semantics=("parallel",))

    y_bf, stats = pl.pallas_call(
        _conv_stats_kernel,
        out_shape=(jax.ShapeDtypeStruct((N, Cin, S), jnp.bfloat16),
                   jax.ShapeDtypeStruct((N, Cin, 2), jnp.float32)),
        grid=(G,),
        in_specs=[
            pl.BlockSpec((_B, Cin, S), lambda n: (n, 0, 0)),
            pl.BlockSpec((Cin, 4 * Cin), lambda n: (0, 0)),
            pl.BlockSpec((Cin, 5 * Cin), lambda n: (0, 0)),
        ],
        out_specs=(pl.BlockSpec((_B, Cin, S), lambda n: (n, 0, 0)),
                   pl.BlockSpec((_B, Cin, 2), lambda n: (n, 0, 0))),
        scratch_shapes=[pltpu.VMEM((_B, 6 * Cin, _FLAT), jnp.bfloat16)],
        compiler_params=parallel,
    )(xs, w3_a, w3_b)

    cnt = float(N * S)
    mean = jnp.sum(stats[:, :, 0], axis=0) / cnt
    var = jnp.maximum(jnp.sum(stats[:, :, 1], axis=0) / cnt - mean * mean,
                      0.0)
    scale = gamma * jax.lax.rsqrt(var + EPS)
    shift = beta - mean * scale

    out_hw = pl.pallas_call(
        _norm_proj_kernel,
        out_shape=jax.ShapeDtypeStruct((N, Cout, S), jnp.float32),
        grid=(G,),
        in_specs=[
            pl.BlockSpec((_B, Cin, S), lambda n: (n, 0, 0)),
            pl.BlockSpec((Cin, 1), lambda n: (0, 0)),
            pl.BlockSpec((Cin, 1), lambda n: (0, 0)),
            pl.BlockSpec((Cout, Cin), lambda n: (0, 0)),
            pl.BlockSpec((Cout, 1), lambda n: (0, 0)),
        ],
        out_specs=pl.BlockSpec((_B, Cout, S), lambda n: (n, 0, 0)),
        compiler_params=parallel,
    )(y_bf, scale.reshape(Cin, 1), shift.reshape(Cin, 1),
      jnp.transpose(wseg_io).astype(jnp.bfloat16), bseg.reshape(Cout, 1))

    return out_hw.reshape(N, Cout, H, W)
